# baseline (speedup 1.0000x reference)
import functools

import jax
import jax.numpy as jnp
from jax import lax
from jax.experimental import pallas as pl
from jax.experimental.pallas import tpu as pltpu
from jax.experimental.pallas import tpu_sc as plsc

D = 32
NC = 2
NS = 16
NW = NC * NS
CHUNK = 1024
IDXW = 128
NIDX = CHUNK // IDXW


@functools.lru_cache(maxsize=None)
def _make_gather(n_rows):
    per_w = n_rows // NW
    n_chunks = per_w // CHUNK
    mesh = plsc.VectorSubcoreMesh(
        core_axis_name="c", subcore_axis_name="s", num_cores=NC, num_subcores=NS
    )

    @functools.partial(
        pl.kernel,
        out_type=jax.ShapeDtypeStruct((n_rows, D), jnp.float32),
        mesh=mesh,
        compiler_params=pltpu.CompilerParams(use_tc_tiling_on_sc=False),
        scratch_types=[
            pltpu.VMEM((2, NIDX, IDXW), jnp.int32),
            pltpu.VMEM((2, CHUNK, D), jnp.float32),
            pltpu.SemaphoreType.DMA,
            pltpu.SemaphoreType.DMA,
            pltpu.SemaphoreType.DMA,
        ],
    )
    def kern(idx_hbm, table_hbm, out_hbm, idx_v, rows_v, isem, gsem, osem):
        wid = lax.axis_index("s") * NC + lax.axis_index("c")
        w_base = wid * per_w

        def idx_copy(g, slot):
            return pltpu.async_copy(
                idx_hbm.at[
                    pl.ds(pl.multiple_of(w_base // IDXW + g * NIDX, 8), NIDX)
                ],
                idx_v.at[slot],
                isem,
            )

        def row_gathers(slot):
            return [
                pltpu.async_copy(
                    table_hbm.at[idx_v.at[slot, j]],
                    rows_v.at[slot, pl.ds(j * IDXW, IDXW)],
                    gsem,
                )
                for j in range(NIDX)
            ]

        idx_copy(0, 0).wait()
        for d in row_gathers(0):
            d.wait()

        @pl.loop(0, n_chunks - 1)
        def _chunk(g):
            slot = lax.rem(g, 2)
            nxt = 1 - slot
            idx_copy(g + 1, nxt).wait()
            nxt_descs = row_gathers(nxt)
            pltpu.async_copy(
                rows_v.at[slot], out_hbm.at[pl.ds(w_base + g * CHUNK, CHUNK)], osem
            ).wait()
            for d in nxt_descs:
                d.wait()

        last = lax.rem(n_chunks - 1, 2)
        pltpu.async_copy(
            rows_v.at[last],
            out_hbm.at[pl.ds(w_base + (n_chunks - 1) * CHUNK, CHUNK)],
            osem,
        ).wait()

    return kern


def _norm_kernel(raw_ref, rclr_ref, out_ref):
    x = raw_ref[...]
    ss = jnp.sum(x * x, axis=1, keepdims=True)
    out_ref[...] = x * (rclr_ref[...] * lax.rsqrt(ss))


@functools.lru_cache(maxsize=None)
def _make_norm(n_rows, blk):
    return pl.pallas_call(
        _norm_kernel,
        grid=(n_rows // blk,),
        in_specs=[
            pl.BlockSpec((blk, D), lambda i: (i, 0)),
            pl.BlockSpec((blk, 1), lambda i: (i, 0)),
        ],
        out_specs=pl.BlockSpec((blk, D), lambda i: (i, 0)),
        out_shape=jax.ShapeDtypeStruct((n_rows, D), jnp.float32),
    )


def kernel(feature, rclr, table):
    b, l = feature.shape
    n = b * l
    idx = feature.reshape(n // IDXW, IDXW).astype(jnp.int32)
    raw = _make_gather(n)(idx, table.astype(jnp.float32))
    out = _make_norm(n, 8192)(raw, rclr.reshape(n, 1))
    return out.reshape(b, l, D)


# speedup vs baseline: 1.3398x; 1.3398x over previous
import functools

import jax
import jax.numpy as jnp
from jax import lax
from jax.experimental import pallas as pl
from jax.experimental.pallas import tpu as pltpu
from jax.experimental.pallas import tpu_sc as plsc

D = 32
NC = 2
NS = 16
NW = NC * NS
CHUNK = 1024
IDXW = 128
NIDX = CHUNK // IDXW


@functools.lru_cache(maxsize=None)
def _make_gather(n_rows):
    per_w = n_rows // NW
    n_chunks = per_w // CHUNK
    mesh = plsc.VectorSubcoreMesh(
        core_axis_name="c", subcore_axis_name="s", num_cores=NC, num_subcores=NS
    )

    @functools.partial(
        pl.kernel,
        out_type=jax.ShapeDtypeStruct((n_rows, D), jnp.float32),
        mesh=mesh,
        compiler_params=pltpu.CompilerParams(use_tc_tiling_on_sc=False),
        scratch_types=[
            pltpu.VMEM((2, NIDX, IDXW), jnp.int32),
            pltpu.VMEM((2, CHUNK, D), jnp.float32),
            pltpu.SemaphoreType.DMA,
            pltpu.SemaphoreType.DMA,
            pltpu.SemaphoreType.DMA,
        ],
    )
    def kern(idx_hbm, table_hbm, out_hbm, idx_v, rows_v, isem, gsem, osem):
        wid = lax.axis_index("s") * NC + lax.axis_index("c")
        w_base = wid * per_w

        def idx_copy(g, slot):
            return pltpu.async_copy(
                idx_hbm.at[
                    pl.ds(pl.multiple_of(w_base // IDXW + g * NIDX, 8), NIDX)
                ],
                idx_v.at[slot],
                isem,
            )

        def row_gathers(slot):
            return [
                pltpu.async_copy(
                    table_hbm.at[idx_v.at[slot, j]],
                    rows_v.at[slot, pl.ds(j * IDXW, IDXW)],
                    gsem,
                )
                for j in range(NIDX)
            ]

        idx_copy(0, 0).wait()
        for d in row_gathers(0):
            d.wait()

        @pl.loop(0, n_chunks - 1)
        def _chunk(g):
            slot = lax.rem(g, 2)
            nxt = 1 - slot
            idx_copy(g + 1, nxt).wait()
            nxt_descs = row_gathers(nxt)
            pltpu.async_copy(
                rows_v.at[slot], out_hbm.at[pl.ds(w_base + g * CHUNK, CHUNK)], osem
            ).wait()
            for d in nxt_descs:
                d.wait()

        last = lax.rem(n_chunks - 1, 2)
        pltpu.async_copy(
            rows_v.at[last],
            out_hbm.at[pl.ds(w_base + (n_chunks - 1) * CHUNK, CHUNK)],
            osem,
        ).wait()

    return kern


def _norm_kernel(bb, l, raw_ref, rclr_ref, out_ref):
    x = raw_ref[...]
    ss = jnp.sum(x * x, axis=1, keepdims=True)
    y = x * lax.rsqrt(ss)
    out_ref[...] = y.reshape(bb, l, D) * rclr_ref[...]


@functools.lru_cache(maxsize=None)
def _make_norm(b, l, bb):
    return pl.pallas_call(
        functools.partial(_norm_kernel, bb, l),
        grid=(b // bb,),
        in_specs=[
            pl.BlockSpec((bb * l, D), lambda i: (i, 0)),
            pl.BlockSpec((bb, l, 1), lambda i: (i, 0, 0)),
        ],
        out_specs=pl.BlockSpec((bb, l, D), lambda i: (i, 0, 0)),
        out_shape=jax.ShapeDtypeStruct((b, l, D), jnp.float32),
    )


def kernel(feature, rclr, table):
    b, l = feature.shape
    n = b * l
    idx = feature.reshape(n // IDXW, IDXW).astype(jnp.int32)
    raw = _make_gather(n)(idx, table.astype(jnp.float32))
    return _make_norm(b, l, 256)(raw, rclr)


# speedup vs baseline: 1.9830x; 1.4800x over previous
import functools

import jax
import jax.numpy as jnp
from jax import lax
from jax.experimental import pallas as pl
from jax.experimental.pallas import tpu as pltpu
from jax.experimental.pallas import tpu_sc as plsc

D = 32
NC = 2
NS = 16
NW = NC * NS
CHUNK = 1024
IDXW = 128
NIDX = CHUNK // IDXW


@functools.lru_cache(maxsize=None)
def _make_gather(n_rows):
    per_w = n_rows // NW
    n_chunks = per_w // CHUNK
    mesh = plsc.VectorSubcoreMesh(
        core_axis_name="c", subcore_axis_name="s", num_cores=NC, num_subcores=NS
    )

    @functools.partial(
        pl.kernel,
        out_type=jax.ShapeDtypeStruct((n_rows, D), jnp.float32),
        mesh=mesh,
        compiler_params=pltpu.CompilerParams(use_tc_tiling_on_sc=False),
        scratch_types=[
            pltpu.VMEM((2, NIDX, IDXW), jnp.int32),
            pltpu.VMEM((2, CHUNK, D), jnp.float32),
            pltpu.SemaphoreType.DMA,
            pltpu.SemaphoreType.DMA,
            pltpu.SemaphoreType.DMA,
        ],
    )
    def kern(idx_hbm, table_hbm, out_hbm, idx_v, rows_v, isem, gsem, osem):
        wid = lax.axis_index("s") * NC + lax.axis_index("c")
        w_base = wid * per_w

        def idx_copy(g, slot):
            return pltpu.async_copy(
                idx_hbm.at[
                    pl.ds(pl.multiple_of(w_base // IDXW + g * NIDX, 8), NIDX)
                ],
                idx_v.at[slot],
                isem,
            )

        def row_gathers(slot):
            return [
                pltpu.async_copy(
                    table_hbm.at[idx_v.at[slot, j]],
                    rows_v.at[slot, pl.ds(j * IDXW, IDXW)],
                    gsem,
                )
                for j in range(NIDX)
            ]

        idx_copy(0, 0).wait()
        for d in row_gathers(0):
            d.wait()

        @pl.loop(0, n_chunks - 1)
        def _chunk(g):
            slot = lax.rem(g, 2)
            nxt = 1 - slot
            idx_copy(g + 1, nxt).wait()
            nxt_descs = row_gathers(nxt)
            pltpu.async_copy(
                rows_v.at[slot], out_hbm.at[pl.ds(w_base + g * CHUNK, CHUNK)], osem
            ).wait()
            for d in nxt_descs:
                d.wait()

        last = lax.rem(n_chunks - 1, 2)
        pltpu.async_copy(
            rows_v.at[last],
            out_hbm.at[pl.ds(w_base + (n_chunks - 1) * CHUNK, CHUNK)],
            osem,
        ).wait()

    return kern


def _norm_kernel(bb, l, raw_ref, rclr_ref, out_ref):
    x = raw_ref[...]
    ss = jnp.sum(x * x, axis=1, keepdims=True)
    y = x * lax.rsqrt(ss)
    yt = y.reshape(bb, l, D).transpose(1, 2, 0)
    out_ref[...] = yt * rclr_ref[...][:, None, :]


@functools.lru_cache(maxsize=None)
def _make_norm(b, l, bb):
    return pl.pallas_call(
        functools.partial(_norm_kernel, bb, l),
        grid=(b // bb,),
        in_specs=[
            pl.BlockSpec((bb * l, D), lambda i: (i, 0)),
            pl.BlockSpec((l, bb), lambda i: (0, i)),
        ],
        out_specs=pl.BlockSpec((l, D, bb), lambda i: (0, 0, i)),
        out_shape=jax.ShapeDtypeStruct((l, D, b), jnp.float32),
    )


def kernel(feature, rclr, table):
    b, l = feature.shape
    n = b * l
    idx = feature.reshape(n // IDXW, IDXW).astype(jnp.int32)
    raw = _make_gather(n)(idx, table.astype(jnp.float32))
    rclr_t = jnp.transpose(rclr, (1, 2, 0)).reshape(l, b)
    out_t = _make_norm(b, l, 256)(raw, rclr_t)
    return jnp.transpose(out_t, (2, 0, 1))


# speedup vs baseline: 2.5502x; 1.2861x over previous
import functools

import jax
import jax.numpy as jnp
from jax import lax
from jax.experimental import pallas as pl
from jax.experimental.pallas import tpu as pltpu
from jax.experimental.pallas import tpu_sc as plsc

D = 32
NC = 2
NS = 16
NW = NC * NS
CHUNK = 1024
IDXW = 128
NIDX = CHUNK // IDXW


@functools.lru_cache(maxsize=None)
def _make_gather(n_rows):
    per_w = n_rows // NW
    n_chunks = per_w // CHUNK
    mesh = plsc.VectorSubcoreMesh(
        core_axis_name="c", subcore_axis_name="s", num_cores=NC, num_subcores=NS
    )

    @functools.partial(
        pl.kernel,
        out_type=jax.ShapeDtypeStruct((n_rows, IDXW), jnp.float32),
        mesh=mesh,
        compiler_params=pltpu.CompilerParams(use_tc_tiling_on_sc=False),
        scratch_types=[
            pltpu.VMEM((2, NIDX, IDXW), jnp.int32),
            pltpu.VMEM((2, CHUNK, D), jnp.float32),
            pltpu.SemaphoreType.DMA,
            pltpu.SemaphoreType.DMA,
            pltpu.SemaphoreType.DMA,
        ],
    )
    def kern(idx_hbm, table_hbm, out_hbm, idx_v, rows_v, isem, gsem, osem):
        wid = lax.axis_index("s") * NC + lax.axis_index("c")
        w_base = wid * per_w

        def idx_copy(g, slot):
            return pltpu.async_copy(
                idx_hbm.at[
                    pl.ds(pl.multiple_of(w_base // IDXW + g * NIDX, 8), NIDX)
                ],
                idx_v.at[slot],
                isem,
            )

        def row_gathers(slot):
            return [
                pltpu.async_copy(
                    table_hbm.at[idx_v.at[slot, j]],
                    rows_v.at[slot, pl.ds(j * IDXW, IDXW)],
                    gsem,
                )
                for j in range(NIDX)
            ]

        idx_copy(0, 0).wait()
        for d in row_gathers(0):
            d.wait()

        @pl.loop(0, n_chunks - 1)
        def _chunk(g):
            slot = lax.rem(g, 2)
            nxt = 1 - slot
            idx_copy(g + 1, nxt).wait()
            nxt_descs = row_gathers(nxt)
            pltpu.async_copy(
                rows_v.at[slot],
                out_hbm.at[pl.ds(w_base + g * CHUNK, CHUNK), pl.ds(0, D)],
                osem,
            ).wait()
            for d in nxt_descs:
                d.wait()

        last = lax.rem(n_chunks - 1, 2)
        pltpu.async_copy(
            rows_v.at[last],
            out_hbm.at[pl.ds(w_base + (n_chunks - 1) * CHUNK, CHUNK), pl.ds(0, D)],
            osem,
        ).wait()

    return kern


def _norm_kernel(bb, l, raw_ref, rclr_ref, out_ref):
    x = raw_ref[:, :D]
    ss = jnp.sum(x * x, axis=1, keepdims=True)
    y = x * lax.rsqrt(ss)
    yt = y.reshape(bb, l, D).transpose(1, 2, 0)
    out_ref[...] = yt * rclr_ref[...][:, None, :]


@functools.lru_cache(maxsize=None)
def _make_norm(b, l, bb):
    return pl.pallas_call(
        functools.partial(_norm_kernel, bb, l),
        grid=(b // bb,),
        in_specs=[
            pl.BlockSpec((bb * l, IDXW), lambda i: (i, 0)),
            pl.BlockSpec((l, bb), lambda i: (0, i)),
        ],
        out_specs=pl.BlockSpec((l, D, bb), lambda i: (0, 0, i)),
        out_shape=jax.ShapeDtypeStruct((l, D, b), jnp.float32),
    )


def kernel(feature, rclr, table):
    b, l = feature.shape
    n = b * l
    idx = feature.reshape(n // IDXW, IDXW).astype(jnp.int32)
    raw = _make_gather(n)(idx, table.astype(jnp.float32))
    rclr_t = jnp.transpose(rclr, (1, 2, 0)).reshape(l, b)
    out_t = _make_norm(b, l, 256)(raw, rclr_t)
    return jnp.transpose(out_t, (2, 0, 1))
